# baseline (speedup 1.0000x reference)
import jax
import jax.numpy as jnp
from jax.experimental import pallas as pl
from jax.experimental.pallas import tpu as pltpu

TOP_K = 5
PROMPT_LEN = 8
POOL = 100
NCH = 8


def _body(task_ref, x_ref, g_ref, ep_ref, ek_ref, cls_ref, out_ref, acc):
    j = pl.program_id(1)
    S_CH = x_ref.shape[1]
    xb = x_ref[0]
    part = jnp.sum(xb, axis=0, keepdims=True)

    @pl.when(j == 0)
    def _():
        acc[...] = part

    @pl.when(j > 0)
    def _():
        acc[...] = acc[...] + part

    pre = (TOP_K + 1) * PROMPT_LEN + 1
    for jj in range(NCH):
        @pl.when(j == jj)
        def _(jj=jj):
            out_ref[0, pre + jj * S_CH:pre + (jj + 1) * S_CH, :] = xb

    @pl.when(j == NCH - 1)
    def _():
        q = acc[...] * (1.0 / (S_CH * NCH))
        qn = q / jnp.maximum(jnp.sqrt(jnp.sum(q * q)), 1e-12)
        ek = ek_ref[...]
        kn = ek / jnp.maximum(
            jnp.sqrt(jnp.sum(ek * ek, axis=1, keepdims=True)), 1e-12)
        sim = jax.lax.dot_general(
            qn, kn, (((1,), (1,)), ((), ())),
            preferred_element_type=jnp.float32)
        tid = task_ref[0]
        out_ref[0, 0:PROMPT_LEN, :] = (
            g_ref[pl.ds(tid * PROMPT_LEN, PROMPT_LEN), :])
        col = jax.lax.broadcasted_iota(jnp.int32, (1, POOL), 1)
        s = sim
        for k in range(TOP_K):
            idx = jnp.argmax(s[0])
            rows = ep_ref[pl.ds(idx * PROMPT_LEN, PROMPT_LEN), :]
            b0 = PROMPT_LEN + k * PROMPT_LEN
            out_ref[0, b0:b0 + PROMPT_LEN, :] = rows
            s = jnp.where(col == idx, -jnp.inf, s)
        ccol = (TOP_K + 1) * PROMPT_LEN
        out_ref[0, ccol:ccol + 1, :] = cls_ref[...]


def kernel(x, g_prompts, e_prompts, e_keys, cls_token, task_id):
    B, S, d = x.shape
    n_out = (TOP_K + 1) * PROMPT_LEN + 1 + S
    ch = S // NCH
    g_flat = g_prompts.reshape(-1, d)
    ep_flat = e_prompts.reshape(-1, d)
    cls2 = cls_token.reshape(1, d)
    task = jnp.asarray(task_id, jnp.int32).reshape(1)
    return pl.pallas_call(
        _body,
        grid=(B, NCH),
        in_specs=[
            pl.BlockSpec(memory_space=pltpu.MemorySpace.SMEM),
            pl.BlockSpec((1, ch, d), lambda b, j: (b, j, 0)),
            pl.BlockSpec(g_flat.shape, lambda b, j: (0, 0)),
            pl.BlockSpec(ep_flat.shape, lambda b, j: (0, 0)),
            pl.BlockSpec(e_keys.shape, lambda b, j: (0, 0)),
            pl.BlockSpec(cls2.shape, lambda b, j: (0, 0)),
        ],
        out_specs=pl.BlockSpec((1, n_out, d), lambda b, j: (b, 0, 0)),
        out_shape=jax.ShapeDtypeStruct((B, n_out, d), x.dtype),
        scratch_shapes=[pltpu.VMEM((1, d), jnp.float32)],
    )(task, x, g_flat, ep_flat, e_keys, cls2)


# speedup vs baseline: 1.2617x; 1.2617x over previous
import jax
import jax.numpy as jnp
from jax.experimental import pallas as pl
from jax.experimental.pallas import tpu as pltpu

TOP_K = 5
PROMPT_LEN = 8
POOL = 100


def _body(task_ref, x_ref, g_ref, ep_ref, ek_ref, cls_ref, out_ref):
    xb = x_ref[0]
    q = jnp.mean(xb, axis=0, keepdims=True)
    qn = q / jnp.maximum(jnp.sqrt(jnp.sum(q * q)), 1e-12)
    ek = ek_ref[...]
    kn = ek / jnp.maximum(
        jnp.sqrt(jnp.sum(ek * ek, axis=1, keepdims=True)), 1e-12)
    sim = jax.lax.dot_general(
        qn, kn, (((1,), (1,)), ((), ())),
        preferred_element_type=jnp.float32)

    tid = task_ref[0]
    out_ref[0, 0:PROMPT_LEN, :] = g_ref[pl.ds(tid * PROMPT_LEN, PROMPT_LEN), :]

    col = jax.lax.broadcasted_iota(jnp.int32, (1, POOL), 1)
    for k in range(TOP_K):
        idx = jnp.argmax(sim[0])
        rows = ep_ref[pl.ds(idx * PROMPT_LEN, PROMPT_LEN), :]
        base = PROMPT_LEN + k * PROMPT_LEN
        out_ref[0, base:base + PROMPT_LEN, :] = rows
        sim = jnp.where(col == idx, -jnp.inf, sim)

    ccol = (TOP_K + 1) * PROMPT_LEN
    out_ref[0, ccol:ccol + 1, :] = cls_ref[...]
    out_ref[0, ccol + 1:, :] = xb


def kernel(x, g_prompts, e_prompts, e_keys, cls_token, task_id):
    B, S, d = x.shape
    n_out = (TOP_K + 1) * PROMPT_LEN + 1 + S
    g_flat = g_prompts.reshape(-1, d)
    ep_flat = e_prompts.reshape(-1, d)
    cls2 = cls_token.reshape(1, d)
    task = jnp.asarray(task_id, jnp.int32).reshape(1)
    return pl.pallas_call(
        _body,
        grid=(B,),
        in_specs=[
            pl.BlockSpec(memory_space=pltpu.MemorySpace.SMEM),
            pl.BlockSpec((1, S, d), lambda b: (b, 0, 0)),
            pl.BlockSpec(g_flat.shape, lambda b: (0, 0)),
            pl.BlockSpec(ep_flat.shape, lambda b: (0, 0)),
            pl.BlockSpec(e_keys.shape, lambda b: (0, 0)),
            pl.BlockSpec(cls2.shape, lambda b: (0, 0)),
        ],
        out_specs=pl.BlockSpec((1, n_out, d), lambda b: (b, 0, 0)),
        out_shape=jax.ShapeDtypeStruct((B, n_out, d), x.dtype),
    )(task, x, g_flat, ep_flat, e_keys, cls2)
